# baseline (speedup 1.0000x reference)
import functools

import numpy as np
import jax
import jax.numpy as jnp
from jax import lax
from jax.experimental import pallas as pl
from jax.experimental.pallas import tpu as pltpu
from jax.experimental.pallas import tpu_sc as plsc

N, D, HD, H = 50000, 256, 1024, 4
RATIO = 0.5
TILE = 1000

R, C = 512, 128
M = R * C
_LANE_JS = [1 << t for t in range(7)]
_ROW_JS = [1 << t for t in range(9)]

_ar_c = np.arange(C)
_P_LANE = np.stack([(_ar_c[:, None] ^ j) == _ar_c[None, :] for j in _LANE_JS]).astype(np.float32)
_ar_r = np.arange(R)
_P_ROW = np.stack([(_ar_r[:, None] ^ j) == _ar_r[None, :] for j in _ROW_JS]).astype(np.float32)


def _score_body(x_ref, w1_ref, b1_ref, w2_ref, b2_ref, out_ref):
    h = jnp.dot(x_ref[...], w1_ref[...]) + b1_ref[...]
    h = jnp.where(h >= 0, h, 0.2 * h)
    out_ref[...] = jnp.dot(h, w2_ref[...]) + b2_ref[...]


def _attn4(x, W1, b1, W2, b2):
    grid = (N // TILE,)
    return pl.pallas_call(
        _score_body,
        grid=grid,
        in_specs=[
            pl.BlockSpec((TILE, D), lambda i: (i, 0)),
            pl.BlockSpec((D, HD), lambda i: (0, 0)),
            pl.BlockSpec((1, HD), lambda i: (0, 0)),
            pl.BlockSpec((HD, H), lambda i: (0, 0)),
            pl.BlockSpec((1, H), lambda i: (0, 0)),
        ],
        out_specs=pl.BlockSpec((TILE, H), lambda i: (i, 0)),
        out_shape=jax.ShapeDtypeStruct((N, H), jnp.float32),
    )(x, W1, b1.reshape(1, HD), W2, b2.reshape(1, H))


def _sort_body(s_ref, i_ref, p_lane_ref, ss_ref, si_ref):
    s = s_ref[...]
    i = i_ref[...]
    r_iota = jax.lax.broadcasted_iota(jnp.int32, (R, C), 0)
    c_iota = jax.lax.broadcasted_iota(jnp.int32, (R, C), 1)
    k = 2
    while k <= M:
        j = k // 2
        while j >= 1:
            if j < C:
                pi = _LANE_JS.index(j)
                both = jnp.concatenate([s, i], axis=0)
                bp = jnp.dot(both, p_lane_ref[pi], precision=jax.lax.Precision.HIGHEST)
                sp, ip = bp[:R], bp[R:]
                ej = (c_iota & j) == 0
            else:
                jr = j // C
                ej = (r_iota & jr) == 0
                s_up = jnp.concatenate([s[jr:], s[:jr]], axis=0)
                s_dn = jnp.concatenate([s[-jr:], s[:-jr]], axis=0)
                sp = jnp.where(ej, s_up, s_dn)
                i_up = jnp.concatenate([i[jr:], i[:jr]], axis=0)
                i_dn = jnp.concatenate([i[-jr:], i[:-jr]], axis=0)
                ip = jnp.where(ej, i_up, i_dn)
            if k < C:
                up = (c_iota & k) == 0
            elif k < M:
                up = (r_iota & (k // C)) == 0
            else:
                up = jnp.full((R, C), True)
            lt = (s > sp) | ((s == sp) & (i < ip))
            keep_e = lt == (up == ej)
            s = jnp.where(keep_e, s, sp)
            i = jnp.where(keep_e, i, ip)
            j //= 2
        k *= 2
    ss_ref[...] = s
    si_ref[...] = i.astype(jnp.int32)


def _topk_sort(scores):
    s2 = jnp.concatenate([scores, jnp.full((M - N,), -1.0, jnp.float32)]).reshape(R, C)
    i2 = jnp.arange(M, dtype=jnp.float32).reshape(R, C)
    ss, si = pl.pallas_call(
        _sort_body,
        out_shape=(
            jax.ShapeDtypeStruct((R, C), jnp.float32),
            jax.ShapeDtypeStruct((R, C), jnp.int32),
        ),
    )(s2, i2, jnp.asarray(_P_LANE))
    return ss.reshape(M), si.reshape(M)


K_KEEP = 25000
NC, NS = 2, 16
NW = NC * NS
Q = 784
CH = 112
NCHUNK = Q // CH


def _gather_scale_body(x_hbm, idx_hbm, ssc_hbm, out_hbm,
                       idx_a, idx_b, rows_a, rows_b, sc_a, sc_b, sem_a, sem_b):
    wid = lax.axis_index("s") * NC + lax.axis_index("c")
    base = wid * Q
    count = jnp.minimum(Q, K_KEEP - base)
    idx_v = [idx_a, idx_b]
    rows_v = [rows_a, rows_b]
    sc_v = [sc_a, sc_b]
    sems = [sem_a, sem_b]

    def launch(c):
        p = c & 1
        start = base + jnp.minimum(c * CH, count - CH)
        pltpu.sync_copy(idx_hbm.at[pl.ds(start, CH)], idx_v[p])
        desc = pltpu.async_copy(x_hbm.at[idx_v[p]], rows_v[p], sems[p])
        pltpu.sync_copy(ssc_hbm.at[pl.ds(start * 16, CH * 16)], sc_v[p])
        return start, desc

    start_c, desc_c = launch(0)
    for c in range(NCHUNK):
        p = c & 1
        desc_c.wait()
        if c + 1 < NCHUNK:
            start_n, desc_n = launch(c + 1)

        def row(r):
            scale = sc_v[p][pl.ds(r * 16, 16)]
            for c16 in range(D // 16):
                sl = pl.ds(c16 * 16, 16)
                rows_v[p][r, sl] = rows_v[p][r, sl] * scale

        plsc.parallel_loop(0, CH, 1, unroll=4)(row)
        pltpu.sync_copy(rows_v[p], out_hbm.at[pl.ds(start_c, CH)])
        if c + 1 < NCHUNK:
            start_c, desc_c = start_n, desc_n


def _gather_scale(x, idx, sorted_scores):
    mesh = plsc.VectorSubcoreMesh(core_axis_name="c", subcore_axis_name="s")
    fn = pl.kernel(
        _gather_scale_body,
        out_type=jax.ShapeDtypeStruct((K_KEEP, D), jnp.float32),
        mesh=mesh,
        scratch_types=[
            pltpu.VMEM((CH,), jnp.int32),
            pltpu.VMEM((CH,), jnp.int32),
            pltpu.VMEM((CH, D), jnp.float32),
            pltpu.VMEM((CH, D), jnp.float32),
            pltpu.VMEM((CH * 16,), jnp.float32),
            pltpu.VMEM((CH * 16,), jnp.float32),
            pltpu.SemaphoreType.DMA,
            pltpu.SemaphoreType.DMA,
        ],
    )
    sc16 = jnp.repeat(1.0 + sorted_scores, 16)
    return fn(x, idx, sc16)


def kernel(x, W1, b1, W2, b2):
    attn4 = _attn4(x, W1, b1, W2, b2)
    attn = attn4.mean(axis=1)
    scores = jax.nn.sigmoid(attn)
    k = max(1, int(RATIO * N))
    sorted_scores, sorted_idx = _topk_sort(scores)
    idx = sorted_idx[:k]
    scaled_feat = _gather_scale(x, idx, sorted_scores[:k])
    return (scaled_feat, idx, scores)


# speedup vs baseline: 1.1102x; 1.1102x over previous
import functools

import numpy as np
import jax
import jax.numpy as jnp
from jax import lax
from jax.experimental import pallas as pl
from jax.experimental.pallas import tpu as pltpu
from jax.experimental.pallas import tpu_sc as plsc

N, D, HD, H = 50000, 256, 1024, 4
RATIO = 0.5
TILE = 1000

R, C = 512, 128
M = R * C
_LANE_JS = [1 << t for t in range(7)]
_ROW_JS = [1 << t for t in range(9)]

_ar_c = np.arange(C)
_P_LANE = np.stack([(_ar_c[:, None] ^ j) == _ar_c[None, :] for j in _LANE_JS]).astype(np.float32)
_ar_r = np.arange(R)
_P_ROW = np.stack([(_ar_r[:, None] ^ j) == _ar_r[None, :] for j in _ROW_JS]).astype(np.float32)


def _score_body(x_ref, w1_ref, b1_ref, w2_ref, b2_ref, out_ref, xs_ref):
    x = x_ref[...]
    h = jnp.dot(x, w1_ref[...]) + b1_ref[...]
    h = jnp.where(h >= 0, h, 0.2 * h)
    a4 = jnp.dot(h, w2_ref[...]) + b2_ref[...]
    out_ref[...] = a4
    m = (a4[:, 0] + a4[:, 1] + a4[:, 2] + a4[:, 3]) * 0.25
    s = 1.0 / (1.0 + jnp.exp(-m))
    xs_ref[...] = x * (1.0 + s)[:, None]


def _attn4(x, W1, b1, W2, b2):
    grid = (N // TILE,)
    return pl.pallas_call(
        _score_body,
        grid=grid,
        in_specs=[
            pl.BlockSpec((TILE, D), lambda i: (i, 0)),
            pl.BlockSpec((D, HD), lambda i: (0, 0)),
            pl.BlockSpec((1, HD), lambda i: (0, 0)),
            pl.BlockSpec((HD, H), lambda i: (0, 0)),
            pl.BlockSpec((1, H), lambda i: (0, 0)),
        ],
        out_specs=[
            pl.BlockSpec((TILE, H), lambda i: (i, 0)),
            pl.BlockSpec((TILE, D), lambda i: (i, 0)),
        ],
        out_shape=(
            jax.ShapeDtypeStruct((N, H), jnp.float32),
            jax.ShapeDtypeStruct((N, D), jnp.float32),
        ),
    )(x, W1, b1.reshape(1, HD), W2, b2.reshape(1, H))


def _sort_body(s_ref, i_ref, p_lane_ref, ss_ref, si_ref):
    s = s_ref[...]
    i = i_ref[...]
    r_iota = jax.lax.broadcasted_iota(jnp.int32, (R, C), 0)
    c_iota = jax.lax.broadcasted_iota(jnp.int32, (R, C), 1)
    k = 2
    while k <= M:
        j = k // 2
        while j >= 1:
            if j < C:
                pi = _LANE_JS.index(j)
                both = jnp.concatenate([s, i], axis=0)
                bp = jnp.dot(both, p_lane_ref[pi], precision=jax.lax.Precision.HIGHEST)
                sp, ip = bp[:R], bp[R:]
                ej = (c_iota & j) == 0
            else:
                jr = j // C
                ej = (r_iota & jr) == 0
                s_up = jnp.concatenate([s[jr:], s[:jr]], axis=0)
                s_dn = jnp.concatenate([s[-jr:], s[:-jr]], axis=0)
                sp = jnp.where(ej, s_up, s_dn)
                i_up = jnp.concatenate([i[jr:], i[:jr]], axis=0)
                i_dn = jnp.concatenate([i[-jr:], i[:-jr]], axis=0)
                ip = jnp.where(ej, i_up, i_dn)
            if k < C:
                up = (c_iota & k) == 0
            elif k < M:
                up = (r_iota & (k // C)) == 0
            else:
                up = jnp.full((R, C), True)
            lt = (s > sp) | ((s == sp) & (i < ip))
            keep_e = lt == (up == ej)
            s = jnp.where(keep_e, s, sp)
            i = jnp.where(keep_e, i, ip)
            j //= 2
        k *= 2
    ss_ref[...] = s
    si_ref[...] = i.astype(jnp.int32)


def _topk_sort(scores):
    s2 = jnp.concatenate([scores, jnp.full((M - N,), -1.0, jnp.float32)]).reshape(R, C)
    i2 = jnp.arange(M, dtype=jnp.float32).reshape(R, C)
    ss, si = pl.pallas_call(
        _sort_body,
        out_shape=(
            jax.ShapeDtypeStruct((R, C), jnp.float32),
            jax.ShapeDtypeStruct((R, C), jnp.int32),
        ),
    )(s2, i2, jnp.asarray(_P_LANE))
    return ss.reshape(M), si.reshape(M)


K_KEEP = 25000
NC, NS = 2, 16
NW = NC * NS
Q = 784
CH = 112
NCHUNK = Q // CH


def _gather_body(xs_hbm, idx_hbm, out_hbm,
                 idx_a, idx_b, rows_a, rows_b, sem_a, sem_b):
    wid = lax.axis_index("s") * NC + lax.axis_index("c")
    base = wid * Q
    count = jnp.minimum(Q, K_KEEP - base)
    idx_v = [idx_a, idx_b]
    rows_v = [rows_a, rows_b]
    sems = [sem_a, sem_b]

    def launch(c):
        p = c & 1
        start = base + jnp.minimum(c * CH, count - CH)
        pltpu.sync_copy(idx_hbm.at[pl.ds(start, CH)], idx_v[p])
        desc = pltpu.async_copy(xs_hbm.at[idx_v[p]], rows_v[p], sems[p])
        return start, desc

    start_c, desc_c = launch(0)
    for c in range(NCHUNK):
        p = c & 1
        desc_c.wait()
        if c + 1 < NCHUNK:
            start_n, desc_n = launch(c + 1)
        pltpu.sync_copy(rows_v[p], out_hbm.at[pl.ds(start_c, CH)])
        if c + 1 < NCHUNK:
            start_c, desc_c = start_n, desc_n


def _gather(xs, idx):
    mesh = plsc.VectorSubcoreMesh(core_axis_name="c", subcore_axis_name="s")
    fn = pl.kernel(
        _gather_body,
        out_type=jax.ShapeDtypeStruct((K_KEEP, D), jnp.float32),
        mesh=mesh,
        scratch_types=[
            pltpu.VMEM((CH,), jnp.int32),
            pltpu.VMEM((CH,), jnp.int32),
            pltpu.VMEM((CH, D), jnp.float32),
            pltpu.VMEM((CH, D), jnp.float32),
            pltpu.SemaphoreType.DMA,
            pltpu.SemaphoreType.DMA,
        ],
    )
    return fn(xs, idx)


def kernel(x, W1, b1, W2, b2):
    attn4, xs = _attn4(x, W1, b1, W2, b2)
    attn = attn4.mean(axis=1)
    scores = jax.nn.sigmoid(attn)
    k = max(1, int(RATIO * N))
    sorted_scores, sorted_idx = _topk_sort(scores)
    idx = sorted_idx[:k]
    scaled_feat = _gather(xs, idx)
    return (scaled_feat, idx, scores)


# speedup vs baseline: 1.1135x; 1.0030x over previous
import functools

import numpy as np
import jax
import jax.numpy as jnp
from jax import lax
from jax.experimental import pallas as pl
from jax.experimental.pallas import tpu as pltpu
from jax.experimental.pallas import tpu_sc as plsc

N, D, HD, H = 50000, 256, 1024, 4
RATIO = 0.5
TILE = 1000

R, C = 512, 128
M = R * C
_LANE_JS = [1 << t for t in range(7)]
_ROW_JS = [1 << t for t in range(9)]

_ar_c = np.arange(C)
_P_LANE = np.stack([(_ar_c[:, None] ^ j) == _ar_c[None, :] for j in _LANE_JS]).astype(np.float32)
_ar_r = np.arange(R)
_P_ROW = np.stack([(_ar_r[:, None] ^ j) == _ar_r[None, :] for j in _ROW_JS]).astype(np.float32)


def _score_body(x_ref, w1_ref, b1_ref, w2_ref, b2_ref, out_ref, xs_ref):
    x = x_ref[...]
    h = jnp.dot(x, w1_ref[...]) + b1_ref[...]
    h = jnp.where(h >= 0, h, 0.2 * h)
    a4 = jnp.dot(h, w2_ref[...]) + b2_ref[...]
    out_ref[...] = a4
    m = (a4[:, 0] + a4[:, 1] + a4[:, 2] + a4[:, 3]) * 0.25
    s = 1.0 / (1.0 + jnp.exp(-m))
    xs_ref[...] = x * (1.0 + s)[:, None]


def _attn4(x, W1, b1, W2, b2):
    grid = (N // TILE,)
    return pl.pallas_call(
        _score_body,
        grid=grid,
        in_specs=[
            pl.BlockSpec((TILE, D), lambda i: (i, 0)),
            pl.BlockSpec((D, HD), lambda i: (0, 0)),
            pl.BlockSpec((1, HD), lambda i: (0, 0)),
            pl.BlockSpec((HD, H), lambda i: (0, 0)),
            pl.BlockSpec((1, H), lambda i: (0, 0)),
        ],
        out_specs=[
            pl.BlockSpec((TILE, H), lambda i: (i, 0)),
            pl.BlockSpec((TILE, D), lambda i: (i, 0)),
        ],
        out_shape=(
            jax.ShapeDtypeStruct((N, H), jnp.float32),
            jax.ShapeDtypeStruct((N, D), jnp.float32),
        ),
    )(x, W1, b1.reshape(1, HD), W2, b2.reshape(1, H))


def _sort_body(s_ref, p_lane_ref, ss_ref, si_ref):
    s = s_ref[...]
    r_iota = jax.lax.broadcasted_iota(jnp.int32, (R, C), 0)
    c_iota = jax.lax.broadcasted_iota(jnp.int32, (R, C), 1)
    i = (r_iota * C + c_iota).astype(jnp.float32)
    k = 2
    while k <= M:
        j = k // 2
        while j >= 1:
            if j < C:
                pi = _LANE_JS.index(j)
                both = jnp.concatenate([s, i], axis=0)
                bp = jnp.dot(both, p_lane_ref[pi], precision=jax.lax.Precision.HIGHEST)
                sp, ip = bp[:R], bp[R:]
                ej = (c_iota & j) == 0
            else:
                jr = j // C
                ej = (r_iota & jr) == 0
                s_up = jnp.concatenate([s[jr:], s[:jr]], axis=0)
                s_dn = jnp.concatenate([s[-jr:], s[:-jr]], axis=0)
                sp = jnp.where(ej, s_up, s_dn)
                i_up = jnp.concatenate([i[jr:], i[:jr]], axis=0)
                i_dn = jnp.concatenate([i[-jr:], i[:-jr]], axis=0)
                ip = jnp.where(ej, i_up, i_dn)
            if k < C:
                up = (c_iota & k) == 0
            elif k < M:
                up = (r_iota & (k // C)) == 0
            else:
                up = jnp.full((R, C), True)
            lt = (s > sp) | ((s == sp) & (i < ip))
            keep_e = lt == (up == ej)
            s = jnp.where(keep_e, s, sp)
            i = jnp.where(keep_e, i, ip)
            j //= 2
        k *= 2
    ss_ref[...] = s
    si_ref[...] = i.astype(jnp.int32)


def _topk_sort(scores):
    s2 = jnp.concatenate([scores, jnp.full((M - N,), -1.0, jnp.float32)]).reshape(R, C)
    ss, si = pl.pallas_call(
        _sort_body,
        out_shape=(
            jax.ShapeDtypeStruct((R, C), jnp.float32),
            jax.ShapeDtypeStruct((R, C), jnp.int32),
        ),
    )(s2, jnp.asarray(_P_LANE))
    return ss.reshape(M), si.reshape(M)


K_KEEP = 25000
NC, NS = 2, 16
NW = NC * NS
Q = 784
CH = 112
NCHUNK = Q // CH


def _gather_body(xs_hbm, idx_hbm, out_hbm,
                 idx_a, idx_b, rows_a, rows_b, sem_a, sem_b):
    wid = lax.axis_index("s") * NC + lax.axis_index("c")
    base = wid * Q
    count = jnp.minimum(Q, K_KEEP - base)
    idx_v = [idx_a, idx_b]
    rows_v = [rows_a, rows_b]
    sems = [sem_a, sem_b]

    def launch(c):
        p = c & 1
        start = base + jnp.minimum(c * CH, count - CH)
        pltpu.sync_copy(idx_hbm.at[pl.ds(start, CH)], idx_v[p])
        desc = pltpu.async_copy(xs_hbm.at[idx_v[p]], rows_v[p], sems[p])
        return start, desc

    start_c, desc_c = launch(0)
    for c in range(NCHUNK):
        p = c & 1
        desc_c.wait()
        if c + 1 < NCHUNK:
            start_n, desc_n = launch(c + 1)
        pltpu.sync_copy(rows_v[p], out_hbm.at[pl.ds(start_c, CH)])
        if c + 1 < NCHUNK:
            start_c, desc_c = start_n, desc_n


def _gather(xs, idx):
    mesh = plsc.VectorSubcoreMesh(core_axis_name="c", subcore_axis_name="s")
    fn = pl.kernel(
        _gather_body,
        out_type=jax.ShapeDtypeStruct((K_KEEP, D), jnp.float32),
        mesh=mesh,
        scratch_types=[
            pltpu.VMEM((CH,), jnp.int32),
            pltpu.VMEM((CH,), jnp.int32),
            pltpu.VMEM((CH, D), jnp.float32),
            pltpu.VMEM((CH, D), jnp.float32),
            pltpu.SemaphoreType.DMA,
            pltpu.SemaphoreType.DMA,
        ],
    )
    return fn(xs, idx)


def kernel(x, W1, b1, W2, b2):
    attn4, xs = _attn4(x, W1, b1, W2, b2)
    attn = attn4.mean(axis=1)
    scores = jax.nn.sigmoid(attn)
    k = max(1, int(RATIO * N))
    sorted_scores, sorted_idx = _topk_sort(scores)
    idx = sorted_idx[:k]
    scaled_feat = _gather(xs, idx)
    return (scaled_feat, idx, scores)
